# initial kernel scaffold (unmeasured)
import jax
import jax.numpy as jnp
from jax import lax
from jax.experimental import pallas as pl
from jax.experimental.pallas import tpu as pltpu

N_DEV = 8
SQ = 512
D = 1024
HL = 8
DH = 128
SKV = 2048
SCALE = 0.08838834764831843
F32 = jnp.float32


def kernel(x, Wq, Wo, K_ext, V_ext):
    x2 = x.reshape(SQ, D)

    def body(x_ref, wq_ref, wo_ref, k_hbm, v_hbm, out_ref,
             k_vmem, v_vmem, xcomm, acc, rs_buf,
             kv_sems, ag_send, ag_recv, rs_send, rs_recv):
        my = lax.axis_index("i")
        right = (my + 1) % N_DEV
        left = (my + N_DEV - 1) % N_DEV
        h0 = my * HL

        kcp = pltpu.make_async_copy(
            k_hbm.at[0, :, pl.ds(h0, HL), :], k_vmem, kv_sems.at[0])
        vcp = pltpu.make_async_copy(
            v_hbm.at[0, :, pl.ds(h0, HL), :], v_vmem, kv_sems.at[1])
        kcp.start()
        vcp.start()

        barrier = pltpu.get_barrier_semaphore()
        for nbr in (left, right):
            pl.semaphore_signal(barrier, inc=1, device_id=(nbr,),
                                device_id_type=pl.DeviceIdType.MESH)
        pl.semaphore_wait(barrier, 2)

        kcp.wait()
        vcp.wait()

        xcomm[pl.ds(my * SQ, SQ), :] = x_ref[:, :]

        def compute_chunk(xc, j):
            q = jnp.dot(xc, wq_ref[:, :], preferred_element_type=F32)
            o_parts = []
            for h in range(HL):
                qh = q[:, h * DH:(h + 1) * DH]
                kh = k_vmem[:, h, :]
                s = lax.dot_general(
                    qh, kh, (((1,), (1,)), ((), ())),
                    preferred_element_type=F32) * SCALE
                m = jnp.max(s, axis=1, keepdims=True)
                p = jnp.exp(s - m)
                l = jnp.sum(p, axis=1, keepdims=True)
                oh = jnp.dot(p, v_vmem[:, h, :],
                             preferred_element_type=F32) / l
                o_parts.append(oh)
            o = jnp.concatenate(o_parts, axis=1)
            acc[pl.ds(j * SQ, SQ), :] = jnp.dot(
                o, wo_ref[:, :], preferred_element_type=F32)

        compute_chunk(x_ref[:, :], my)

        for h in range(N_DEV - 1):
            j_send = (my - h) % N_DEV
            rdma = pltpu.make_async_remote_copy(
                src_ref=xcomm.at[pl.ds(j_send * SQ, SQ), :],
                dst_ref=xcomm.at[pl.ds(j_send * SQ, SQ), :],
                send_sem=ag_send.at[h],
                recv_sem=ag_recv.at[h],
                device_id=(right,),
                device_id_type=pl.DeviceIdType.MESH,
            )
            rdma.start()
            rdma.wait()
            j_recv = (my - h - 1) % N_DEV
            compute_chunk(xcomm[pl.ds(j_recv * SQ, SQ), :], j_recv)

        for s in range(N_DEV - 1):
            c = (my - s - 1) % N_DEV
            if s > 0:
                acc[pl.ds(c * SQ, SQ), :] = (
                    acc[pl.ds(c * SQ, SQ), :] + rs_buf[s - 1, :, :])
            rdma = pltpu.make_async_remote_copy(
                src_ref=acc.at[pl.ds(c * SQ, SQ), :],
                dst_ref=rs_buf.at[s],
                send_sem=rs_send.at[s],
                recv_sem=rs_recv.at[s],
                device_id=(right,),
                device_id_type=pl.DeviceIdType.MESH,
            )
            rdma.start()
            rdma.wait()

        out_ref[:, :] = acc[pl.ds(my * SQ, SQ), :] + rs_buf[N_DEV - 2, :, :]

    out = pl.pallas_call(
        body,
        out_shape=jax.ShapeDtypeStruct((SQ, D), F32),
        in_specs=[
            pl.BlockSpec(memory_space=pltpu.VMEM),
            pl.BlockSpec(memory_space=pltpu.VMEM),
            pl.BlockSpec(memory_space=pltpu.VMEM),
            pl.BlockSpec(memory_space=pltpu.ANY),
            pl.BlockSpec(memory_space=pltpu.ANY),
        ],
        out_specs=pl.BlockSpec(memory_space=pltpu.VMEM),
        scratch_shapes=[
            pltpu.VMEM((SKV, HL, DH), F32),
            pltpu.VMEM((SKV, HL, DH), F32),
            pltpu.VMEM((N_DEV * SQ, D), F32),
            pltpu.VMEM((N_DEV * SQ, D), F32),
            pltpu.VMEM((N_DEV - 1, SQ, D), F32),
            pltpu.SemaphoreType.DMA((2,)),
            pltpu.SemaphoreType.DMA((N_DEV - 1,)),
            pltpu.SemaphoreType.DMA((N_DEV - 1,)),
            pltpu.SemaphoreType.DMA((N_DEV - 1,)),
            pltpu.SemaphoreType.DMA((N_DEV - 1,)),
        ],
        compiler_params=pltpu.CompilerParams(
            collective_id=0,
            vmem_limit_bytes=112 * 1024 * 1024,
        ),
    )(x2, Wq, Wo, K_ext, V_ext)
    return out.reshape(1, SQ, D)


# baseline (device time: 462028 ns/iter reference)
import jax
import jax.numpy as jnp
from jax import lax
from jax.experimental import pallas as pl
from jax.experimental.pallas import tpu as pltpu

N_DEV = 8
SQ = 512
D = 1024
HL = 8
DH = 128
SKV = 2048
SCALE = 0.08838834764831843
F32 = jnp.float32


def kernel(x, Wq, Wo, K_ext, V_ext):
    x2 = x.reshape(SQ, D)
    wq3 = Wq.reshape(D, HL, DH).transpose(1, 0, 2)
    wo3 = Wo.reshape(HL, DH, D)

    def body(x_ref, wq_ref, wo_ref, k_hbm, v_hbm, out_ref,
             k_vmem, v_vmem, xslot, rs_send_buf, rs_recv_buf,
             pacc, kv_sems, x_send_sems, x_recv_sems, rs_send_sems,
             rs_recv_sems, x_credit, rs_credit):
        my = lax.axis_index("i")
        right = (my + 1) % N_DEV
        left = (my + N_DEV - 1) % N_DEV
        h0 = my * HL

        kv_copies = []
        for h in range(HL):
            kcp = pltpu.make_async_copy(
                k_hbm.at[0, :, h0 + h, :], k_vmem.at[h], kv_sems.at[h])
            vcp = pltpu.make_async_copy(
                v_hbm.at[0, :, h0 + h, :], v_vmem.at[h], kv_sems.at[HL + h])
            kcp.start()
            vcp.start()
            kv_copies.append((kcp, vcp))

        barrier = pltpu.get_barrier_semaphore()
        for nbr in (left, right):
            pl.semaphore_signal(barrier, inc=1, device_id=(nbr,),
                                device_id_type=pl.DeviceIdType.MESH)
        pl.semaphore_wait(barrier, 2)

        for kcp, vcp in kv_copies:
            kcp.wait()
            vcp.wait()

        def compute_chunk(src_ref, src_off, dst_ref, dst_off):
            pacc[:, :] = jnp.zeros((SQ, D), F32)

            def head_step(h, carry):
                xc = src_ref[pl.ds(src_off, SQ), :]
                qh = jnp.dot(xc, wq_ref[h], preferred_element_type=F32)
                s = lax.dot_general(
                    qh, k_vmem[h], (((1,), (1,)), ((), ())),
                    preferred_element_type=F32) * SCALE
                m = jnp.max(s, axis=1, keepdims=True)
                p = jnp.exp(s - m)
                l = jnp.sum(p, axis=1, keepdims=True)
                oh = jnp.dot(p, v_vmem[h],
                             preferred_element_type=F32) / l
                pacc[:, :] = pacc[:, :] + jnp.dot(
                    oh, wo_ref[h], preferred_element_type=F32)
                return carry

            lax.fori_loop(0, HL, head_step, None)
            dst_ref[pl.ds(dst_off, SQ), :] = pacc[:, :]

        def xslot_at(p):
            return xslot.at[pl.ds(p * SQ, SQ), :]

        def rs_send_at(p):
            return rs_send_buf.at[pl.ds(p * SQ, SQ), :]

        def rs_recv_at(p):
            return rs_recv_buf.at[pl.ds(p * SQ, SQ), :]

        xslot[pl.ds(0, SQ), :] = x_ref[:, :]
        compute_chunk(x_ref, 0, out_ref, 0)

        def step(t, _):
            sp = (t - 1) % 2
            dp = t % 2

            @pl.when(t >= 2)
            def _():
                pl.semaphore_wait(x_credit.at[dp], 1)

            xr = pltpu.make_async_remote_copy(
                src_ref=xslot_at(sp),
                dst_ref=xslot_at(dp),
                send_sem=x_send_sems.at[sp],
                recv_sem=x_recv_sems.at[dp],
                device_id=(right,),
                device_id_type=pl.DeviceIdType.MESH,
            )
            xr.start()

            @pl.when(t >= 2)
            def _():
                @pl.when(t >= 4)
                def _():
                    pl.semaphore_wait(rs_credit.at[dp], 1)

                @pl.when(t >= 3)
                def _():
                    rs_in = pltpu.make_async_remote_copy(
                        src_ref=rs_send_at(sp),
                        dst_ref=rs_recv_at(sp),
                        send_sem=rs_send_sems.at[sp],
                        recv_sem=rs_recv_sems.at[sp],
                        device_id=(left,),
                        device_id_type=pl.DeviceIdType.MESH,
                    )
                    rs_in.wait_recv()
                    rs_send_buf[pl.ds(sp * SQ, SQ), :] = (
                        rs_send_buf[pl.ds(sp * SQ, SQ), :]
                        + rs_recv_buf[pl.ds(sp * SQ, SQ), :])
                    pl.semaphore_signal(rs_credit.at[sp], inc=1,
                                        device_id=(left,),
                                        device_id_type=pl.DeviceIdType.MESH)

                rs_out = pltpu.make_async_remote_copy(
                    src_ref=rs_send_at(sp),
                    dst_ref=rs_recv_at(dp),
                    send_sem=rs_send_sems.at[sp],
                    recv_sem=rs_recv_sems.at[dp],
                    device_id=(right,),
                    device_id_type=pl.DeviceIdType.MESH,
                )
                rs_out.start()

            xr.wait_recv()
            compute_chunk(xslot, dp * SQ, rs_send_buf, dp * SQ)

            xr.wait_send()

            @pl.when(t <= 6)
            def _():
                pl.semaphore_signal(x_credit.at[sp], inc=1,
                                    device_id=(left,),
                                    device_id_type=pl.DeviceIdType.MESH)

            @pl.when(t >= 2)
            def _():
                rs_done = pltpu.make_async_remote_copy(
                    src_ref=rs_send_at(sp),
                    dst_ref=rs_recv_at(dp),
                    send_sem=rs_send_sems.at[sp],
                    recv_sem=rs_recv_sems.at[dp],
                    device_id=(right,),
                    device_id_type=pl.DeviceIdType.MESH,
                )
                rs_done.wait_send()

            return _

        lax.fori_loop(1, N_DEV, step, None)

        pl.semaphore_wait(rs_credit.at[0], 1)
        rs_in = pltpu.make_async_remote_copy(
            src_ref=rs_send_at(1), dst_ref=rs_recv_at(1),
            send_sem=rs_send_sems.at[1], recv_sem=rs_recv_sems.at[1],
            device_id=(left,), device_id_type=pl.DeviceIdType.MESH,
        )
        rs_in.wait_recv()
        rs_send_buf[pl.ds(SQ, SQ), :] = (
            rs_send_buf[pl.ds(SQ, SQ), :] + rs_recv_buf[pl.ds(SQ, SQ), :])
        rs_out = pltpu.make_async_remote_copy(
            src_ref=rs_send_at(1), dst_ref=rs_recv_at(0),
            send_sem=rs_send_sems.at[1], recv_sem=rs_recv_sems.at[0],
            device_id=(right,), device_id_type=pl.DeviceIdType.MESH,
        )
        rs_out.start()
        rs_out.wait_recv()
        out_ref[:, :] = out_ref[:, :] + rs_recv_buf[pl.ds(0, SQ), :]
        rs_out.wait_send()

    out = pl.pallas_call(
        body,
        out_shape=jax.ShapeDtypeStruct((SQ, D), F32),
        in_specs=[
            pl.BlockSpec(memory_space=pltpu.MemorySpace.VMEM),
            pl.BlockSpec(memory_space=pltpu.MemorySpace.VMEM),
            pl.BlockSpec(memory_space=pltpu.MemorySpace.VMEM),
            pl.BlockSpec(memory_space=pl.ANY),
            pl.BlockSpec(memory_space=pl.ANY),
        ],
        out_specs=pl.BlockSpec(memory_space=pltpu.MemorySpace.VMEM),
        scratch_shapes=[
            pltpu.VMEM((HL, SKV, DH), F32),
            pltpu.VMEM((HL, SKV, DH), F32),
            pltpu.VMEM((2 * SQ, D), F32),
            pltpu.VMEM((2 * SQ, D), F32),
            pltpu.VMEM((2 * SQ, D), F32),
            pltpu.VMEM((SQ, D), F32),
            pltpu.SemaphoreType.DMA((2 * HL,)),
            pltpu.SemaphoreType.DMA((2,)),
            pltpu.SemaphoreType.DMA((2,)),
            pltpu.SemaphoreType.DMA((2,)),
            pltpu.SemaphoreType.DMA((2,)),
            pltpu.SemaphoreType.REGULAR((2,)),
            pltpu.SemaphoreType.REGULAR((2,)),
        ],
        compiler_params=pltpu.CompilerParams(
            collective_id=0,
            vmem_limit_bytes=60 * 1024 * 1024,
        ),
    )(x2, wq3, wo3, K_ext, V_ext)
    return out.reshape(1, SQ, D)


# device time: 457704 ns/iter; 1.0094x vs baseline; 1.0094x over previous
import jax
import jax.numpy as jnp
from jax import lax
from jax.experimental import pallas as pl
from jax.experimental.pallas import tpu as pltpu

N_DEV = 8
SQ = 512
D = 1024
HL = 8
DH = 128
SKV = 2048
SCALE = 0.08838834764831843
F32 = jnp.float32
BF16 = jnp.bfloat16


def kernel(x, Wq, Wo, K_ext, V_ext):
    x2 = x.reshape(SQ, D).astype(BF16)
    wq3 = Wq.reshape(D, HL, DH).transpose(1, 0, 2).astype(BF16)
    wo3 = Wo.reshape(HL, DH, D).astype(BF16)
    kb = K_ext[0].transpose(1, 0, 2).astype(BF16)
    vb = V_ext[0].transpose(1, 0, 2).astype(BF16)

    def body(x_ref, wq_ref, wo_ref, k_hbm, v_hbm, out_ref,
             k_vmem, v_vmem, xslot, rs_send_buf, rs_recv_buf,
             pacc, kv_sems, x_send_sems, x_recv_sems, rs_send_sems,
             rs_recv_sems, x_credit, rs_credit):
        my = lax.axis_index("i")
        right = (my + 1) % N_DEV
        left = (my + N_DEV - 1) % N_DEV
        h0 = my * HL

        kcp = pltpu.make_async_copy(
            k_hbm.at[pl.ds(h0, HL)], k_vmem, kv_sems.at[0])
        vcp = pltpu.make_async_copy(
            v_hbm.at[pl.ds(h0, HL)], v_vmem, kv_sems.at[1])
        kcp.start()
        vcp.start()

        barrier = pltpu.get_barrier_semaphore()
        for nbr in (left, right):
            pl.semaphore_signal(barrier, inc=1, device_id=(nbr,),
                                device_id_type=pl.DeviceIdType.MESH)
        pl.semaphore_wait(barrier, 2)

        kcp.wait()
        vcp.wait()

        def compute_chunk(src_ref, src_off, dst_ref, dst_off):
            pacc[:, :] = jnp.zeros((SQ, D), F32)

            def head_step(h, carry):
                xc = src_ref[pl.ds(src_off, SQ), :]
                qh = jnp.dot(xc, wq_ref[h], preferred_element_type=F32)
                s = lax.dot_general(
                    qh.astype(BF16), k_vmem[h], (((1,), (1,)), ((), ())),
                    preferred_element_type=F32) * SCALE
                m = jnp.max(s, axis=1, keepdims=True)
                p = jnp.exp(s - m)
                l = jnp.sum(p, axis=1, keepdims=True)
                oh = jnp.dot(p.astype(BF16), v_vmem[h],
                             preferred_element_type=F32) / l
                pacc[:, :] = pacc[:, :] + jnp.dot(
                    oh.astype(BF16), wo_ref[h], preferred_element_type=F32)
                return carry

            lax.fori_loop(0, HL, head_step, None)
            dst_ref[pl.ds(dst_off, SQ), :] = pacc[:, :]

        def xslot_at(p):
            return xslot.at[pl.ds(p * SQ, SQ), :]

        def rs_send_at(p):
            return rs_send_buf.at[pl.ds(p * SQ, SQ), :]

        def rs_recv_at(p):
            return rs_recv_buf.at[pl.ds(p * SQ, SQ), :]

        xslot[pl.ds(0, SQ), :] = x_ref[:, :]
        compute_chunk(x_ref, 0, out_ref, 0)

        def step(t, _):
            sp = (t - 1) % 2
            dp = t % 2

            @pl.when(t >= 2)
            def _():
                pl.semaphore_wait(x_credit.at[dp], 1)

            xr = pltpu.make_async_remote_copy(
                src_ref=xslot_at(sp),
                dst_ref=xslot_at(dp),
                send_sem=x_send_sems.at[sp],
                recv_sem=x_recv_sems.at[dp],
                device_id=(right,),
                device_id_type=pl.DeviceIdType.MESH,
            )
            xr.start()

            @pl.when(t >= 2)
            def _():
                @pl.when(t >= 4)
                def _():
                    pl.semaphore_wait(rs_credit.at[dp], 1)

                @pl.when(t >= 3)
                def _():
                    rs_in = pltpu.make_async_remote_copy(
                        src_ref=rs_send_at(sp),
                        dst_ref=rs_recv_at(sp),
                        send_sem=rs_send_sems.at[sp],
                        recv_sem=rs_recv_sems.at[sp],
                        device_id=(left,),
                        device_id_type=pl.DeviceIdType.MESH,
                    )
                    rs_in.wait_recv()
                    rs_send_buf[pl.ds(sp * SQ, SQ), :] = (
                        rs_send_buf[pl.ds(sp * SQ, SQ), :]
                        + rs_recv_buf[pl.ds(sp * SQ, SQ), :])
                    pl.semaphore_signal(rs_credit.at[sp], inc=1,
                                        device_id=(left,),
                                        device_id_type=pl.DeviceIdType.MESH)

                rs_out = pltpu.make_async_remote_copy(
                    src_ref=rs_send_at(sp),
                    dst_ref=rs_recv_at(dp),
                    send_sem=rs_send_sems.at[sp],
                    recv_sem=rs_recv_sems.at[dp],
                    device_id=(right,),
                    device_id_type=pl.DeviceIdType.MESH,
                )
                rs_out.start()

            xr.wait_recv()
            compute_chunk(xslot, dp * SQ, rs_send_buf, dp * SQ)

            xr.wait_send()

            @pl.when(t <= 6)
            def _():
                pl.semaphore_signal(x_credit.at[sp], inc=1,
                                    device_id=(left,),
                                    device_id_type=pl.DeviceIdType.MESH)

            @pl.when(t >= 2)
            def _():
                rs_done = pltpu.make_async_remote_copy(
                    src_ref=rs_send_at(sp),
                    dst_ref=rs_recv_at(dp),
                    send_sem=rs_send_sems.at[sp],
                    recv_sem=rs_recv_sems.at[dp],
                    device_id=(right,),
                    device_id_type=pl.DeviceIdType.MESH,
                )
                rs_done.wait_send()

            return _

        lax.fori_loop(1, N_DEV, step, None)

        pl.semaphore_wait(rs_credit.at[0], 1)
        rs_in = pltpu.make_async_remote_copy(
            src_ref=rs_send_at(1), dst_ref=rs_recv_at(1),
            send_sem=rs_send_sems.at[1], recv_sem=rs_recv_sems.at[1],
            device_id=(left,), device_id_type=pl.DeviceIdType.MESH,
        )
        rs_in.wait_recv()
        rs_send_buf[pl.ds(SQ, SQ), :] = (
            rs_send_buf[pl.ds(SQ, SQ), :] + rs_recv_buf[pl.ds(SQ, SQ), :])
        rs_out = pltpu.make_async_remote_copy(
            src_ref=rs_send_at(1), dst_ref=rs_recv_at(0),
            send_sem=rs_send_sems.at[1], recv_sem=rs_recv_sems.at[0],
            device_id=(right,), device_id_type=pl.DeviceIdType.MESH,
        )
        rs_out.start()
        rs_out.wait_recv()
        out_ref[:, :] = out_ref[:, :] + rs_recv_buf[pl.ds(0, SQ), :]
        rs_out.wait_send()

    out = pl.pallas_call(
        body,
        out_shape=jax.ShapeDtypeStruct((SQ, D), F32),
        in_specs=[
            pl.BlockSpec(memory_space=pltpu.MemorySpace.VMEM),
            pl.BlockSpec(memory_space=pltpu.MemorySpace.VMEM),
            pl.BlockSpec(memory_space=pltpu.MemorySpace.VMEM),
            pl.BlockSpec(memory_space=pl.ANY),
            pl.BlockSpec(memory_space=pl.ANY),
        ],
        out_specs=pl.BlockSpec(memory_space=pltpu.MemorySpace.VMEM),
        scratch_shapes=[
            pltpu.VMEM((HL, SKV, DH), BF16),
            pltpu.VMEM((HL, SKV, DH), BF16),
            pltpu.VMEM((2 * SQ, D), BF16),
            pltpu.VMEM((2 * SQ, D), F32),
            pltpu.VMEM((2 * SQ, D), F32),
            pltpu.VMEM((SQ, D), F32),
            pltpu.SemaphoreType.DMA((2,)),
            pltpu.SemaphoreType.DMA((2,)),
            pltpu.SemaphoreType.DMA((2,)),
            pltpu.SemaphoreType.DMA((2,)),
            pltpu.SemaphoreType.DMA((2,)),
            pltpu.SemaphoreType.REGULAR((2,)),
            pltpu.SemaphoreType.REGULAR((2,)),
        ],
        compiler_params=pltpu.CompilerParams(
            collective_id=0,
            vmem_limit_bytes=60 * 1024 * 1024,
        ),
    )(x2, wq3, wo3, kb, vb)
    return out.reshape(1, SQ, D)


# device time: 375646 ns/iter; 1.2300x vs baseline; 1.2184x over previous
import jax
import jax.numpy as jnp
from jax import lax
from jax.experimental import pallas as pl
from jax.experimental.pallas import tpu as pltpu

N_DEV = 8
SQ = 512
D = 1024
HL = 8
DH = 128
SKV = 2048
SCALE = 0.08838834764831843
F32 = jnp.float32
BF16 = jnp.bfloat16


def kernel(x, Wq, Wo, K_ext, V_ext):
    x2 = x.reshape(SQ, D).astype(BF16)
    wq3 = Wq.reshape(D, HL, DH).transpose(1, 0, 2).astype(BF16)
    wo3 = Wo.reshape(HL, DH, D).astype(BF16)

    def body(x_ref, wq_ref, wo_ref, k_hbm, v_hbm, out_ref,
             k_vmem, v_vmem, k_f32, v_f32, xslot, rs_send_buf, rs_recv_buf,
             pacc, kv_sems, x_send_sems, x_recv_sems, rs_send_sems,
             rs_recv_sems, x_credit, rs_credit):
        my = lax.axis_index("i")
        right = (my + 1) % N_DEV
        left = (my + N_DEV - 1) % N_DEV
        h0 = my * HL

        kv_copies = []
        for h in range(HL):
            kcp = pltpu.make_async_copy(
                k_hbm.at[0, :, h0 + h, :], k_f32.at[h], kv_sems.at[h])
            vcp = pltpu.make_async_copy(
                v_hbm.at[0, :, h0 + h, :], v_f32.at[h], kv_sems.at[HL + h])
            kcp.start()
            vcp.start()
            kv_copies.append((kcp, vcp))

        barrier = pltpu.get_barrier_semaphore()
        for nbr in (left, right):
            pl.semaphore_signal(barrier, inc=1, device_id=(nbr,),
                                device_id_type=pl.DeviceIdType.MESH)
        pl.semaphore_wait(barrier, 2)

        for kcp, vcp in kv_copies:
            kcp.wait()
            vcp.wait()
        k_vmem[:, :, :] = k_f32[:, :, :].astype(BF16)
        v_vmem[:, :, :] = v_f32[:, :, :].astype(BF16)

        def compute_chunk(src_ref, src_off, dst_ref, dst_off):
            pacc[:, :] = jnp.zeros((SQ, D), F32)

            def head_step(h, carry):
                xc = src_ref[pl.ds(src_off, SQ), :]
                qh = jnp.dot(xc, wq_ref[h], preferred_element_type=F32)
                s = lax.dot_general(
                    qh.astype(BF16), k_vmem[h], (((1,), (1,)), ((), ())),
                    preferred_element_type=F32) * SCALE
                m = jnp.max(s, axis=1, keepdims=True)
                p = jnp.exp(s - m)
                l = jnp.sum(p, axis=1, keepdims=True)
                oh = jnp.dot(p.astype(BF16), v_vmem[h],
                             preferred_element_type=F32) / l
                pacc[:, :] = pacc[:, :] + jnp.dot(
                    oh.astype(BF16), wo_ref[h], preferred_element_type=F32)
                return carry

            lax.fori_loop(0, HL, head_step, None)
            dst_ref[pl.ds(dst_off, SQ), :] = pacc[:, :].astype(dst_ref.dtype)

        def xslot_at(p):
            return xslot.at[pl.ds(p * SQ, SQ), :]

        def rs_send_at(p):
            return rs_send_buf.at[pl.ds(p * SQ, SQ), :]

        def rs_recv_at(p):
            return rs_recv_buf.at[pl.ds(p * SQ, SQ), :]

        xslot[pl.ds(0, SQ), :] = x_ref[:, :]
        compute_chunk(x_ref, 0, out_ref, 0)

        def step(t, _):
            sp = (t - 1) % 2
            dp = t % 2

            @pl.when(t >= 2)
            def _():
                pl.semaphore_wait(x_credit.at[dp], 1)

            xr = pltpu.make_async_remote_copy(
                src_ref=xslot_at(sp),
                dst_ref=xslot_at(dp),
                send_sem=x_send_sems.at[sp],
                recv_sem=x_recv_sems.at[dp],
                device_id=(right,),
                device_id_type=pl.DeviceIdType.MESH,
            )
            xr.start()

            @pl.when(t >= 2)
            def _():
                @pl.when(t >= 4)
                def _():
                    pl.semaphore_wait(rs_credit.at[dp], 1)

                @pl.when(t >= 3)
                def _():
                    rs_in = pltpu.make_async_remote_copy(
                        src_ref=rs_send_at(sp),
                        dst_ref=rs_recv_at(sp),
                        send_sem=rs_send_sems.at[sp],
                        recv_sem=rs_recv_sems.at[sp],
                        device_id=(left,),
                        device_id_type=pl.DeviceIdType.MESH,
                    )
                    rs_in.wait_recv()
                    rs_send_buf[pl.ds(sp * SQ, SQ), :] = (
                        rs_send_buf[pl.ds(sp * SQ, SQ), :]
                        + rs_recv_buf[pl.ds(sp * SQ, SQ), :])
                    pl.semaphore_signal(rs_credit.at[sp], inc=1,
                                        device_id=(left,),
                                        device_id_type=pl.DeviceIdType.MESH)

                rs_out = pltpu.make_async_remote_copy(
                    src_ref=rs_send_at(sp),
                    dst_ref=rs_recv_at(dp),
                    send_sem=rs_send_sems.at[sp],
                    recv_sem=rs_recv_sems.at[dp],
                    device_id=(right,),
                    device_id_type=pl.DeviceIdType.MESH,
                )
                rs_out.start()

            xr.wait_recv()
            compute_chunk(xslot, dp * SQ, rs_send_buf, dp * SQ)

            xr.wait_send()

            @pl.when(t <= 6)
            def _():
                pl.semaphore_signal(x_credit.at[sp], inc=1,
                                    device_id=(left,),
                                    device_id_type=pl.DeviceIdType.MESH)

            @pl.when(t >= 2)
            def _():
                rs_done = pltpu.make_async_remote_copy(
                    src_ref=rs_send_at(sp),
                    dst_ref=rs_recv_at(dp),
                    send_sem=rs_send_sems.at[sp],
                    recv_sem=rs_recv_sems.at[dp],
                    device_id=(right,),
                    device_id_type=pl.DeviceIdType.MESH,
                )
                rs_done.wait_send()

            return _

        lax.fori_loop(1, N_DEV, step, None)

        pl.semaphore_wait(rs_credit.at[0], 1)
        rs_in = pltpu.make_async_remote_copy(
            src_ref=rs_send_at(1), dst_ref=rs_recv_at(1),
            send_sem=rs_send_sems.at[1], recv_sem=rs_recv_sems.at[1],
            device_id=(left,), device_id_type=pl.DeviceIdType.MESH,
        )
        rs_in.wait_recv()
        rs_send_buf[pl.ds(SQ, SQ), :] = (
            rs_send_buf[pl.ds(SQ, SQ), :] + rs_recv_buf[pl.ds(SQ, SQ), :])
        rs_out = pltpu.make_async_remote_copy(
            src_ref=rs_send_at(1), dst_ref=rs_recv_at(0),
            send_sem=rs_send_sems.at[1], recv_sem=rs_recv_sems.at[0],
            device_id=(right,), device_id_type=pl.DeviceIdType.MESH,
        )
        rs_out.start()
        rs_out.wait_recv()
        out_ref[:, :] = out_ref[:, :] + rs_recv_buf[pl.ds(0, SQ), :]
        rs_out.wait_send()

    out = pl.pallas_call(
        body,
        out_shape=jax.ShapeDtypeStruct((SQ, D), F32),
        in_specs=[
            pl.BlockSpec(memory_space=pltpu.MemorySpace.VMEM),
            pl.BlockSpec(memory_space=pltpu.MemorySpace.VMEM),
            pl.BlockSpec(memory_space=pltpu.MemorySpace.VMEM),
            pl.BlockSpec(memory_space=pl.ANY),
            pl.BlockSpec(memory_space=pl.ANY),
        ],
        out_specs=pl.BlockSpec(memory_space=pltpu.MemorySpace.VMEM),
        scratch_shapes=[
            pltpu.VMEM((HL, SKV, DH), BF16),
            pltpu.VMEM((HL, SKV, DH), BF16),
            pltpu.VMEM((HL, SKV, DH), F32),
            pltpu.VMEM((HL, SKV, DH), F32),
            pltpu.VMEM((2 * SQ, D), BF16),
            pltpu.VMEM((2 * SQ, D), BF16),
            pltpu.VMEM((2 * SQ, D), BF16),
            pltpu.VMEM((SQ, D), F32),
            pltpu.SemaphoreType.DMA((2 * HL,)),
            pltpu.SemaphoreType.DMA((2,)),
            pltpu.SemaphoreType.DMA((2,)),
            pltpu.SemaphoreType.DMA((2,)),
            pltpu.SemaphoreType.DMA((2,)),
            pltpu.SemaphoreType.REGULAR((2,)),
            pltpu.SemaphoreType.REGULAR((2,)),
        ],
        compiler_params=pltpu.CompilerParams(
            collective_id=0,
            vmem_limit_bytes=60 * 1024 * 1024,
        ),
    )(x2, wq3, wo3, K_ext, V_ext)
    return out.reshape(1, SQ, D)


# device time: 298684 ns/iter; 1.5469x vs baseline; 1.2577x over previous
import jax
import jax.numpy as jnp
from jax import lax
from jax.experimental import pallas as pl
from jax.experimental.pallas import tpu as pltpu

N_DEV = 8
SQ = 512
D = 1024
HL = 8
DH = 128
SKV = 2048
SCALE = 0.08838834764831843
F32 = jnp.float32
BF16 = jnp.bfloat16


def kernel(x, Wq, Wo, K_ext, V_ext):
    x2 = x.reshape(SQ, D).astype(BF16)
    wq3 = (Wq * SCALE).reshape(D, HL, DH).transpose(1, 0, 2).astype(BF16)
    wo3 = Wo.reshape(HL, DH, D).astype(BF16)

    def body(x_ref, wq_ref, wo_ref, k_hbm, v_hbm, out_ref,
             k_vmem, v_vmem, k_f32, v_f32, xslot, rs_send_buf, rs_recv_buf,
             pacc, kv_sems, x_send_sems, x_recv_sems, rs_send_sems,
             rs_recv_sems, x_credit, rs_credit):
        my = lax.axis_index("i")
        right = (my + 1) % N_DEV
        left = (my + N_DEV - 1) % N_DEV
        h0 = my * HL

        kv_copies = []
        for h in range(HL):
            kcp = pltpu.make_async_copy(
                k_hbm.at[0, :, h0 + h, :], k_f32.at[h], kv_sems.at[h])
            vcp = pltpu.make_async_copy(
                v_hbm.at[0, :, h0 + h, :], v_f32.at[h], kv_sems.at[HL + h])
            kcp.start()
            vcp.start()
            kv_copies.append((kcp, vcp))

        barrier = pltpu.get_barrier_semaphore()
        for nbr in (left, right):
            pl.semaphore_signal(barrier, inc=1, device_id=(nbr,),
                                device_id_type=pl.DeviceIdType.MESH)
        pl.semaphore_wait(barrier, 2)

        for kcp, vcp in kv_copies:
            kcp.wait()
            vcp.wait()
        k_vmem[:, :, :] = k_f32[:, :, :].astype(BF16)
        v_vmem[:, :, :] = v_f32[:, :, :].astype(BF16)

        def compute_chunk(src_ref, src_off, dst_ref, dst_off):
            pacc[:, :] = jnp.zeros((SQ, D), F32)

            def head_step(h, carry):
                xc = src_ref[pl.ds(src_off, SQ), :]
                qh = jnp.dot(xc, wq_ref[h], preferred_element_type=F32)
                s = lax.dot_general(
                    qh.astype(BF16), k_vmem[h], (((1,), (1,)), ((), ())),
                    preferred_element_type=F32)
                p = jnp.exp(s)
                l = jnp.sum(p, axis=1, keepdims=True)
                oh = jnp.dot(p.astype(BF16), v_vmem[h],
                             preferred_element_type=F32) / l
                pacc[:, :] = pacc[:, :] + jnp.dot(
                    oh.astype(BF16), wo_ref[h], preferred_element_type=F32)
                return carry

            lax.fori_loop(0, HL, head_step, None)
            dst_ref[pl.ds(dst_off, SQ), :] = pacc[:, :].astype(dst_ref.dtype)

        def xslot_at(p):
            return xslot.at[pl.ds(p * SQ, SQ), :]

        def rs_send_at(p):
            return rs_send_buf.at[pl.ds(p * SQ, SQ), :]

        def rs_recv_at(p):
            return rs_recv_buf.at[pl.ds(p * SQ, SQ), :]

        xslot[pl.ds(0, SQ), :] = x_ref[:, :]
        compute_chunk(x_ref, 0, out_ref, 0)

        def step(t, _):
            sp = (t - 1) % 2
            dp = t % 2

            @pl.when(t >= 2)
            def _():
                pl.semaphore_wait(x_credit.at[dp], 1)

            xr = pltpu.make_async_remote_copy(
                src_ref=xslot_at(sp),
                dst_ref=xslot_at(dp),
                send_sem=x_send_sems.at[sp],
                recv_sem=x_recv_sems.at[dp],
                device_id=(right,),
                device_id_type=pl.DeviceIdType.MESH,
            )
            xr.start()

            @pl.when(t >= 2)
            def _():
                @pl.when(t >= 4)
                def _():
                    pl.semaphore_wait(rs_credit.at[dp], 1)

                @pl.when(t >= 3)
                def _():
                    rs_in = pltpu.make_async_remote_copy(
                        src_ref=rs_send_at(sp),
                        dst_ref=rs_recv_at(sp),
                        send_sem=rs_send_sems.at[sp],
                        recv_sem=rs_recv_sems.at[sp],
                        device_id=(left,),
                        device_id_type=pl.DeviceIdType.MESH,
                    )
                    rs_in.wait_recv()
                    rs_send_buf[pl.ds(sp * SQ, SQ), :] = (
                        rs_send_buf[pl.ds(sp * SQ, SQ), :]
                        + rs_recv_buf[pl.ds(sp * SQ, SQ), :])
                    pl.semaphore_signal(rs_credit.at[sp], inc=1,
                                        device_id=(left,),
                                        device_id_type=pl.DeviceIdType.MESH)

                rs_out = pltpu.make_async_remote_copy(
                    src_ref=rs_send_at(sp),
                    dst_ref=rs_recv_at(dp),
                    send_sem=rs_send_sems.at[sp],
                    recv_sem=rs_recv_sems.at[dp],
                    device_id=(right,),
                    device_id_type=pl.DeviceIdType.MESH,
                )
                rs_out.start()

            xr.wait_recv()
            compute_chunk(xslot, dp * SQ, rs_send_buf, dp * SQ)

            xr.wait_send()

            @pl.when(t <= 6)
            def _():
                pl.semaphore_signal(x_credit.at[sp], inc=1,
                                    device_id=(left,),
                                    device_id_type=pl.DeviceIdType.MESH)

            @pl.when(t >= 2)
            def _():
                rs_done = pltpu.make_async_remote_copy(
                    src_ref=rs_send_at(sp),
                    dst_ref=rs_recv_at(dp),
                    send_sem=rs_send_sems.at[sp],
                    recv_sem=rs_recv_sems.at[dp],
                    device_id=(right,),
                    device_id_type=pl.DeviceIdType.MESH,
                )
                rs_done.wait_send()

            return _

        lax.fori_loop(1, N_DEV, step, None)

        pl.semaphore_wait(rs_credit.at[0], 1)
        rs_in = pltpu.make_async_remote_copy(
            src_ref=rs_send_at(1), dst_ref=rs_recv_at(1),
            send_sem=rs_send_sems.at[1], recv_sem=rs_recv_sems.at[1],
            device_id=(left,), device_id_type=pl.DeviceIdType.MESH,
        )
        rs_in.wait_recv()
        rs_send_buf[pl.ds(SQ, SQ), :] = (
            rs_send_buf[pl.ds(SQ, SQ), :] + rs_recv_buf[pl.ds(SQ, SQ), :])
        rs_out = pltpu.make_async_remote_copy(
            src_ref=rs_send_at(1), dst_ref=rs_recv_at(0),
            send_sem=rs_send_sems.at[1], recv_sem=rs_recv_sems.at[0],
            device_id=(right,), device_id_type=pl.DeviceIdType.MESH,
        )
        rs_out.start()
        rs_out.wait_recv()
        out_ref[:, :] = out_ref[:, :] + rs_recv_buf[pl.ds(0, SQ), :]
        rs_out.wait_send()

    out = pl.pallas_call(
        body,
        out_shape=jax.ShapeDtypeStruct((SQ, D), F32),
        in_specs=[
            pl.BlockSpec(memory_space=pltpu.MemorySpace.VMEM),
            pl.BlockSpec(memory_space=pltpu.MemorySpace.VMEM),
            pl.BlockSpec(memory_space=pltpu.MemorySpace.VMEM),
            pl.BlockSpec(memory_space=pl.ANY),
            pl.BlockSpec(memory_space=pl.ANY),
        ],
        out_specs=pl.BlockSpec(memory_space=pltpu.MemorySpace.VMEM),
        scratch_shapes=[
            pltpu.VMEM((HL, SKV, DH), BF16),
            pltpu.VMEM((HL, SKV, DH), BF16),
            pltpu.VMEM((HL, SKV, DH), F32),
            pltpu.VMEM((HL, SKV, DH), F32),
            pltpu.VMEM((2 * SQ, D), BF16),
            pltpu.VMEM((2 * SQ, D), BF16),
            pltpu.VMEM((2 * SQ, D), BF16),
            pltpu.VMEM((SQ, D), F32),
            pltpu.SemaphoreType.DMA((2 * HL,)),
            pltpu.SemaphoreType.DMA((2,)),
            pltpu.SemaphoreType.DMA((2,)),
            pltpu.SemaphoreType.DMA((2,)),
            pltpu.SemaphoreType.DMA((2,)),
            pltpu.SemaphoreType.REGULAR((2,)),
            pltpu.SemaphoreType.REGULAR((2,)),
        ],
        compiler_params=pltpu.CompilerParams(
            collective_id=0,
            vmem_limit_bytes=60 * 1024 * 1024,
        ),
    )(x2, wq3, wo3, K_ext, V_ext)
    return out.reshape(1, SQ, D)
